# Initial kernel scaffold; baseline (speedup 1.0000x reference)
import jax
import jax.numpy as jnp
from jax.experimental import pallas as pl


def kernel(x, edge_index, W1, b1, W2, b2):
    raise NotImplementedError("write your pallas kernel here")


# speedup vs baseline: 12.0508x; 12.0508x over previous
import functools

import jax
import jax.numpy as jnp
from jax import lax
from jax.experimental import pallas as pl
from jax.experimental.pallas import tpu as pltpu
from jax.experimental.pallas import tpu_sc as plsc

_N = 10000
_E = 160000
_F = 128
_NC = 2
_NS = 16
_W = _NC * _NS

_EPT_DEG = _E // _W
_DCH = 128
_NDCH = _EPT_DEG // _DCH
_EPT = _E // _NS
_CH = 128
_NCHUNK = _EPT // _CH
_TAIL = _EPT - _NCHUNK * _CH
_RPT = 9984 // _NS
_RTAIL = _N - _RPT * _NS

_mesh = plsc.VectorSubcoreMesh(core_axis_name="c", subcore_axis_name="s")


@functools.partial(
    pl.kernel,
    mesh=_mesh,
    out_type=[
        jax.ShapeDtypeStruct((_N, 16), jnp.float32),
        jax.ShapeDtypeStruct((_N, 16), jnp.float32),
    ],
    scratch_types=[
        pltpu.VMEM((_DCH,), jnp.int32),
        pltpu.VMEM((16,), jnp.int32),
        pltpu.VMEM((_DCH, 16), jnp.float32),
        pltpu.VMEM((16, 16), jnp.float32),
        pltpu.VMEM((_DCH, 16), jnp.float32),
        pltpu.VMEM_SHARED((_N, 16), jnp.float32),
    ],
)
def _deg_kernel(dst_hbm, out0_hbm, out1_hbm, dst_v, dstt_v, ones_v, tail_v,
                zeros_v, acc_sh):
    c = lax.axis_index("c")
    s = lax.axis_index("s")

    zero16f = jnp.zeros((16,), jnp.float32)
    ones16f = jnp.ones((16,), jnp.float32)

    def fbody(i, carry):
        ones_v[i, pl.ds(0, 16)] = ones16f
        zeros_v[i, pl.ds(0, 16)] = zero16f
        return carry

    lax.fori_loop(0, _DCH, fbody, 0)
    for i in range(16):
        tail_v[i, pl.ds(0, 16)] = zero16f if i < 8 else ones16f

    r0 = s * _RPT
    for k in range(_RPT // _DCH):
        pltpu.sync_copy(zeros_v, acc_sh.at[pl.ds(r0 + k * _DCH, _DCH)])
    rem = _RPT % _DCH
    pltpu.sync_copy(zeros_v.at[pl.ds(0, rem)],
                    acc_sh.at[pl.ds(r0 + (_RPT // _DCH) * _DCH, rem)])

    @pl.when(s == 0)
    def _():
        pltpu.sync_copy(zeros_v.at[pl.ds(0, _RTAIL)],
                        acc_sh.at[pl.ds(_RPT * _NS, _RTAIL)])

    plsc.subcore_barrier()

    ebase = (c * _NS + s) * _EPT_DEG

    def body(i, carry):
        pltpu.sync_copy(dst_hbm.at[pl.ds(ebase + i * _DCH, _DCH)], dst_v)
        pltpu.sync_copy(ones_v, acc_sh.at[dst_v], add=True)
        return carry

    lax.fori_loop(0, _NDCH, body, 0)
    pltpu.sync_copy(dst_hbm.at[pl.ds(ebase + _EPT_DEG - 16, 16)], dstt_v)
    pltpu.sync_copy(tail_v, acc_sh.at[dstt_v], add=True)

    plsc.subcore_barrier()

    @pl.when(c == 0)
    def _():
        pltpu.sync_copy(acc_sh.at[pl.ds(r0, _RPT)],
                        out0_hbm.at[pl.ds(r0, _RPT)])

        @pl.when(s == 0)
        def _():
            pltpu.sync_copy(acc_sh.at[pl.ds(_RPT * _NS, _RTAIL)],
                            out0_hbm.at[pl.ds(_RPT * _NS, _RTAIL)])

    @pl.when(c == 1)
    def _():
        pltpu.sync_copy(acc_sh.at[pl.ds(r0, _RPT)],
                        out1_hbm.at[pl.ds(r0, _RPT)])

        @pl.when(s == 0)
        def _():
            pltpu.sync_copy(acc_sh.at[pl.ds(_RPT * _NS, _RTAIL)],
                            out1_hbm.at[pl.ds(_RPT * _NS, _RTAIL)])


@functools.partial(
    pl.kernel,
    mesh=_mesh,
    out_type=[
        jax.ShapeDtypeStruct((_N, _F), jnp.float32),
        jax.ShapeDtypeStruct((_N, _F), jnp.float32),
    ],
    scratch_types=[
        pltpu.VMEM((_CH,), jnp.int32),
        pltpu.VMEM((_CH,), jnp.int32),
        pltpu.VMEM((_CH, _F), jnp.float32),
        pltpu.VMEM((16,), jnp.int32),
        pltpu.VMEM((16,), jnp.int32),
        pltpu.VMEM((16, _F), jnp.float32),
        pltpu.VMEM_SHARED((_N, _F), jnp.float32),
        pltpu.SemaphoreType.DMA,
    ],
)
def _scatter_kernel(ulo_hbm, uhi_hbm, src_hbm, dst_hbm, outlo_hbm, outhi_hbm,
                    src_v, dst_v, rows_v, srct_v, dstt_v, rowst_v, acc_sh,
                    sem):
    c = lax.axis_index("c")
    s = lax.axis_index("s")

    zero16f = jnp.zeros((16,), jnp.float32)

    def zbody(i, carry):
        for j in range(_F // 16):
            rows_v[i, pl.ds(j * 16, 16)] = zero16f
        return carry

    lax.fori_loop(0, _CH, zbody, 0)

    r0 = s * _RPT
    for k in range(_RPT // _CH):
        pltpu.sync_copy(rows_v, acc_sh.at[pl.ds(r0 + k * _CH, _CH)])
    rem = _RPT % _CH
    pltpu.sync_copy(rows_v.at[pl.ds(0, rem)],
                    acc_sh.at[pl.ds(r0 + (_RPT // _CH) * _CH, rem)])

    @pl.when(s == 0)
    def _():
        pltpu.sync_copy(rows_v.at[pl.ds(0, _RTAIL)],
                        acc_sh.at[pl.ds(_RPT * _NS, _RTAIL)])

    plsc.subcore_barrier()

    ebase = s * _EPT

    def body(i, carry):
        off = ebase + i * _CH
        pltpu.sync_copy(src_hbm.at[pl.ds(off, _CH)], src_v)
        pltpu.sync_copy(dst_hbm.at[pl.ds(off, _CH)], dst_v)

        @pl.when(c == 0)
        def _():
            pltpu.async_copy(ulo_hbm.at[src_v], rows_v, sem).wait()

        @pl.when(c == 1)
        def _():
            pltpu.async_copy(uhi_hbm.at[src_v], rows_v, sem).wait()

        pltpu.sync_copy(rows_v, acc_sh.at[dst_v], add=True)
        return carry

    lax.fori_loop(0, _NCHUNK, body, 0)

    toff = ebase + _NCHUNK * _CH
    pltpu.sync_copy(src_hbm.at[pl.ds(toff, _TAIL)], srct_v)
    pltpu.sync_copy(dst_hbm.at[pl.ds(toff, _TAIL)], dstt_v)

    @pl.when(c == 0)
    def _():
        pltpu.async_copy(ulo_hbm.at[srct_v], rowst_v, sem).wait()

    @pl.when(c == 1)
    def _():
        pltpu.async_copy(uhi_hbm.at[srct_v], rowst_v, sem).wait()

    pltpu.sync_copy(rowst_v, acc_sh.at[dstt_v], add=True)

    plsc.subcore_barrier()

    @pl.when(c == 0)
    def _():
        pltpu.sync_copy(acc_sh.at[pl.ds(r0, _RPT)],
                        outlo_hbm.at[pl.ds(r0, _RPT)])

        @pl.when(s == 0)
        def _():
            pltpu.sync_copy(acc_sh.at[pl.ds(_RPT * _NS, _RTAIL)],
                            outlo_hbm.at[pl.ds(_RPT * _NS, _RTAIL)])

    @pl.when(c == 1)
    def _():
        pltpu.sync_copy(acc_sh.at[pl.ds(r0, _RPT)],
                        outhi_hbm.at[pl.ds(r0, _RPT)])

        @pl.when(s == 0)
        def _():
            pltpu.sync_copy(acc_sh.at[pl.ds(_RPT * _NS, _RTAIL)],
                            outhi_hbm.at[pl.ds(_RPT * _NS, _RTAIL)])


_R = 512
_GRID = (_N + _R - 1) // _R


def _prep_body(deg0_ref, deg1_ref, x_ref, ulo_ref, uhi_ref, dinv_ref):
    deg = deg0_ref[:, 0:1] + deg1_ref[:, 0:1] + 1.0
    dinv = lax.rsqrt(deg)
    dinv_ref[...] = dinv
    ulo_ref[...] = x_ref[:, :_F] * dinv
    uhi_ref[...] = x_ref[:, _F:] * dinv


def _prep_call(deg0, deg1, x):
    return pl.pallas_call(
        _prep_body,
        grid=(_GRID,),
        in_specs=[
            pl.BlockSpec((_R, 16), lambda r: (r, 0)),
            pl.BlockSpec((_R, 16), lambda r: (r, 0)),
            pl.BlockSpec((_R, 2 * _F), lambda r: (r, 0)),
        ],
        out_specs=[
            pl.BlockSpec((_R, _F), lambda r: (r, 0)),
            pl.BlockSpec((_R, _F), lambda r: (r, 0)),
            pl.BlockSpec((_R, 1), lambda r: (r, 0)),
        ],
        out_shape=[
            jax.ShapeDtypeStruct((_N, _F), jnp.float32),
            jax.ShapeDtypeStruct((_N, _F), jnp.float32),
            jax.ShapeDtypeStruct((_N, 1), jnp.float32),
        ],
    )(deg0, deg1, x)


def _mid_body(slo_ref, shi_ref, ulo_ref, uhi_ref, dinv_ref, w1_ref, b1_ref,
              w2_ref, olo_ref, ohi_ref):
    dv = dinv_ref[...]
    alo = (slo_ref[...] + ulo_ref[...]) * dv
    ahi = (shi_ref[...] + uhi_ref[...]) * dv
    h = jnp.dot(alo, w1_ref[:_F, :], preferred_element_type=jnp.float32)
    h = h + jnp.dot(ahi, w1_ref[_F:, :], preferred_element_type=jnp.float32)
    h = jnp.maximum(h + b1_ref[...], 0.0)
    g = jnp.dot(h, w2_ref[...], preferred_element_type=jnp.float32)
    olo_ref[...] = g[:, :_F] * dv
    ohi_ref[...] = g[:, _F:] * dv


def _mid_call(slo, shi, ulo, uhi, dinv, W1, b1, W2):
    return pl.pallas_call(
        _mid_body,
        grid=(_GRID,),
        in_specs=[
            pl.BlockSpec((_R, _F), lambda r: (r, 0)),
            pl.BlockSpec((_R, _F), lambda r: (r, 0)),
            pl.BlockSpec((_R, _F), lambda r: (r, 0)),
            pl.BlockSpec((_R, _F), lambda r: (r, 0)),
            pl.BlockSpec((_R, 1), lambda r: (r, 0)),
            pl.BlockSpec((2 * _F, 512), lambda r: (0, 0)),
            pl.BlockSpec((1, 512), lambda r: (0, 0)),
            pl.BlockSpec((512, 2 * _F), lambda r: (0, 0)),
        ],
        out_specs=[
            pl.BlockSpec((_R, _F), lambda r: (r, 0)),
            pl.BlockSpec((_R, _F), lambda r: (r, 0)),
        ],
        out_shape=[
            jax.ShapeDtypeStruct((_N, _F), jnp.float32),
            jax.ShapeDtypeStruct((_N, _F), jnp.float32),
        ],
    )(slo, shi, ulo, uhi, dinv, W1, b1, W2)


def _final_body(slo_ref, shi_ref, ulo_ref, uhi_ref, dinv_ref, b2_ref,
                out_ref):
    dv = dinv_ref[...]
    out_ref[:, :_F] = jax.nn.sigmoid((slo_ref[...] + ulo_ref[...]) * dv
                                     + b2_ref[:, :_F])
    out_ref[:, _F:] = jax.nn.sigmoid((shi_ref[...] + uhi_ref[...]) * dv
                                     + b2_ref[:, _F:])


def _final_call(slo, shi, ulo, uhi, dinv, b2):
    return pl.pallas_call(
        _final_body,
        grid=(_GRID,),
        in_specs=[
            pl.BlockSpec((_R, _F), lambda r: (r, 0)),
            pl.BlockSpec((_R, _F), lambda r: (r, 0)),
            pl.BlockSpec((_R, _F), lambda r: (r, 0)),
            pl.BlockSpec((_R, _F), lambda r: (r, 0)),
            pl.BlockSpec((_R, 1), lambda r: (r, 0)),
            pl.BlockSpec((1, 2 * _F), lambda r: (0, 0)),
        ],
        out_specs=pl.BlockSpec((_R, 2 * _F), lambda r: (r, 0)),
        out_shape=jax.ShapeDtypeStruct((_N, 2 * _F), jnp.float32),
    )(slo, shi, ulo, uhi, dinv, b2)


def kernel(x, edge_index, W1, b1, W2, b2):
    src = edge_index[0].astype(jnp.int32)
    dst = edge_index[1].astype(jnp.int32)

    deg0, deg1 = _deg_kernel(dst)
    ulo, uhi, dinv = _prep_call(deg0, deg1, x)
    s1lo, s1hi = _scatter_kernel(ulo, uhi, src, dst)
    u2lo, u2hi = _mid_call(s1lo, s1hi, ulo, uhi, dinv, W1,
                           b1.reshape(1, -1), W2)
    s2lo, s2hi = _scatter_kernel(u2lo, u2hi, src, dst)
    return _final_call(s2lo, s2hi, u2lo, u2hi, dinv, b2.reshape(1, -1))


# speedup vs baseline: 17.4833x; 1.4508x over previous
import functools

import jax
import jax.numpy as jnp
from jax import lax
from jax.experimental import pallas as pl
from jax.experimental.pallas import tpu as pltpu
from jax.experimental.pallas import tpu_sc as plsc

_N = 10000
_E = 160000
_F = 128
_NC = 2
_NS = 16
_W = _NC * _NS

_EPT_DEG = _E // _W
_DCH = 128
_NDCH = _EPT_DEG // _DCH
_EPT = _E // _NS
_CH = 128
_NCHUNK = _EPT // _CH
_TAIL = _EPT - _NCHUNK * _CH
_RPT = 9984 // _NS
_RTAIL = _N - _RPT * _NS

_mesh = plsc.VectorSubcoreMesh(core_axis_name="c", subcore_axis_name="s")


@functools.partial(
    pl.kernel,
    mesh=_mesh,
    out_type=[
        jax.ShapeDtypeStruct((_N, 16), jnp.float32),
        jax.ShapeDtypeStruct((_N, 16), jnp.float32),
    ],
    scratch_types=[
        pltpu.VMEM((_DCH,), jnp.int32),
        pltpu.VMEM((16,), jnp.int32),
        pltpu.VMEM((_DCH, 16), jnp.float32),
        pltpu.VMEM((16, 16), jnp.float32),
        pltpu.VMEM((_DCH, 16), jnp.float32),
        pltpu.VMEM_SHARED((_N, 16), jnp.float32),
    ],
)
def _deg_kernel(dst_hbm, out0_hbm, out1_hbm, dst_v, dstt_v, ones_v, tail_v,
                zeros_v, acc_sh):
    c = lax.axis_index("c")
    s = lax.axis_index("s")

    zero16f = jnp.zeros((16,), jnp.float32)
    ones16f = jnp.ones((16,), jnp.float32)

    def fbody(i, carry):
        ones_v[i, pl.ds(0, 16)] = ones16f
        zeros_v[i, pl.ds(0, 16)] = zero16f
        return carry

    lax.fori_loop(0, _DCH, fbody, 0)
    for i in range(16):
        tail_v[i, pl.ds(0, 16)] = zero16f if i < 8 else ones16f

    r0 = s * _RPT
    for k in range(_RPT // _DCH):
        pltpu.sync_copy(zeros_v, acc_sh.at[pl.ds(r0 + k * _DCH, _DCH)])
    rem = _RPT % _DCH
    pltpu.sync_copy(zeros_v.at[pl.ds(0, rem)],
                    acc_sh.at[pl.ds(r0 + (_RPT // _DCH) * _DCH, rem)])

    @pl.when(s == 0)
    def _():
        pltpu.sync_copy(zeros_v.at[pl.ds(0, _RTAIL)],
                        acc_sh.at[pl.ds(_RPT * _NS, _RTAIL)])

    plsc.subcore_barrier()

    ebase = (c * _NS + s) * _EPT_DEG

    def body(i, carry):
        pltpu.sync_copy(dst_hbm.at[pl.ds(ebase + i * _DCH, _DCH)], dst_v)
        pltpu.sync_copy(ones_v, acc_sh.at[dst_v], add=True)
        return carry

    lax.fori_loop(0, _NDCH, body, 0)
    pltpu.sync_copy(dst_hbm.at[pl.ds(ebase + _EPT_DEG - 16, 16)], dstt_v)
    pltpu.sync_copy(tail_v, acc_sh.at[dstt_v], add=True)

    plsc.subcore_barrier()

    @pl.when(c == 0)
    def _():
        pltpu.sync_copy(acc_sh.at[pl.ds(r0, _RPT)],
                        out0_hbm.at[pl.ds(r0, _RPT)])

        @pl.when(s == 0)
        def _():
            pltpu.sync_copy(acc_sh.at[pl.ds(_RPT * _NS, _RTAIL)],
                            out0_hbm.at[pl.ds(_RPT * _NS, _RTAIL)])

    @pl.when(c == 1)
    def _():
        pltpu.sync_copy(acc_sh.at[pl.ds(r0, _RPT)],
                        out1_hbm.at[pl.ds(r0, _RPT)])

        @pl.when(s == 0)
        def _():
            pltpu.sync_copy(acc_sh.at[pl.ds(_RPT * _NS, _RTAIL)],
                            out1_hbm.at[pl.ds(_RPT * _NS, _RTAIL)])


@functools.partial(
    pl.kernel,
    mesh=_mesh,
    out_type=[
        jax.ShapeDtypeStruct((_N, _F), jnp.float32),
        jax.ShapeDtypeStruct((_N, _F), jnp.float32),
    ],
    scratch_types=[
        pltpu.VMEM((_CH,), jnp.int32),
        pltpu.VMEM((_CH,), jnp.int32),
        pltpu.VMEM((_CH, _F), jnp.float32),
        pltpu.VMEM((_CH,), jnp.int32),
        pltpu.VMEM((_CH,), jnp.int32),
        pltpu.VMEM((_CH, _F), jnp.float32),
        pltpu.VMEM((16,), jnp.int32),
        pltpu.VMEM((16,), jnp.int32),
        pltpu.VMEM((16, _F), jnp.float32),
        pltpu.VMEM_SHARED((_N, _F), jnp.float32),
        pltpu.SemaphoreType.DMA,
        pltpu.SemaphoreType.DMA,
    ],
)
def _scatter_kernel(ulo_hbm, uhi_hbm, src_hbm, dst_hbm, outlo_hbm, outhi_hbm,
                    src_v0, dst_v0, rows_v0, src_v1, dst_v1, rows_v1,
                    srct_v, dstt_v, rowst_v, acc_sh, sem0, sem1):
    c = lax.axis_index("c")
    s = lax.axis_index("s")

    def _fire(idx_ref, rows_ref, sem):
        @pl.when(c == 0)
        def _():
            pltpu.async_copy(ulo_hbm.at[idx_ref], rows_ref, sem)

        @pl.when(c == 1)
        def _():
            pltpu.async_copy(uhi_hbm.at[idx_ref], rows_ref, sem)

    def _drain(idx_ref, rows_ref, sem):
        @pl.when(c == 0)
        def _():
            pltpu.make_async_copy(ulo_hbm.at[idx_ref], rows_ref, sem).wait()

        @pl.when(c == 1)
        def _():
            pltpu.make_async_copy(uhi_hbm.at[idx_ref], rows_ref, sem).wait()

    zero16f = jnp.zeros((16,), jnp.float32)

    def zbody(i, carry):
        for j in range(_F // 16):
            rows_v0[i, pl.ds(j * 16, 16)] = zero16f
        return carry

    lax.fori_loop(0, _CH, zbody, 0)

    r0 = s * _RPT
    for k in range(_RPT // _CH):
        pltpu.sync_copy(rows_v0, acc_sh.at[pl.ds(r0 + k * _CH, _CH)])
    rem = _RPT % _CH
    pltpu.sync_copy(rows_v0.at[pl.ds(0, rem)],
                    acc_sh.at[pl.ds(r0 + (_RPT // _CH) * _CH, rem)])

    @pl.when(s == 0)
    def _():
        pltpu.sync_copy(rows_v0.at[pl.ds(0, _RTAIL)],
                        acc_sh.at[pl.ds(_RPT * _NS, _RTAIL)])

    plsc.subcore_barrier()

    ebase = s * _EPT

    pltpu.sync_copy(src_hbm.at[pl.ds(ebase, _CH)], src_v0)
    pltpu.sync_copy(dst_hbm.at[pl.ds(ebase, _CH)], dst_v0)
    _fire(src_v0, rows_v0, sem0)

    def body(k, carry):
        off1 = ebase + (2 * k + 1) * _CH
        pltpu.sync_copy(src_hbm.at[pl.ds(off1, _CH)], src_v1)
        pltpu.sync_copy(dst_hbm.at[pl.ds(off1, _CH)], dst_v1)
        _fire(src_v1, rows_v1, sem1)

        _drain(src_v0, rows_v0, sem0)
        pltpu.sync_copy(rows_v0, acc_sh.at[dst_v0], add=True)

        @pl.when(k < _NCHUNK // 2 - 1)
        def _():
            off0 = ebase + (2 * k + 2) * _CH
            pltpu.sync_copy(src_hbm.at[pl.ds(off0, _CH)], src_v0)
            pltpu.sync_copy(dst_hbm.at[pl.ds(off0, _CH)], dst_v0)
            _fire(src_v0, rows_v0, sem0)

        _drain(src_v1, rows_v1, sem1)
        pltpu.sync_copy(rows_v1, acc_sh.at[dst_v1], add=True)
        return carry

    lax.fori_loop(0, _NCHUNK // 2, body, 0)

    toff = ebase + _NCHUNK * _CH
    pltpu.sync_copy(src_hbm.at[pl.ds(toff, _TAIL)], srct_v)
    pltpu.sync_copy(dst_hbm.at[pl.ds(toff, _TAIL)], dstt_v)

    _fire(srct_v, rowst_v, sem0)
    _drain(srct_v, rowst_v, sem0)

    pltpu.sync_copy(rowst_v, acc_sh.at[dstt_v], add=True)

    plsc.subcore_barrier()

    @pl.when(c == 0)
    def _():
        pltpu.sync_copy(acc_sh.at[pl.ds(r0, _RPT)],
                        outlo_hbm.at[pl.ds(r0, _RPT)])

        @pl.when(s == 0)
        def _():
            pltpu.sync_copy(acc_sh.at[pl.ds(_RPT * _NS, _RTAIL)],
                            outlo_hbm.at[pl.ds(_RPT * _NS, _RTAIL)])

    @pl.when(c == 1)
    def _():
        pltpu.sync_copy(acc_sh.at[pl.ds(r0, _RPT)],
                        outhi_hbm.at[pl.ds(r0, _RPT)])

        @pl.when(s == 0)
        def _():
            pltpu.sync_copy(acc_sh.at[pl.ds(_RPT * _NS, _RTAIL)],
                            outhi_hbm.at[pl.ds(_RPT * _NS, _RTAIL)])


_R = 512
_GRID = (_N + _R - 1) // _R


def _prep_body(deg0_ref, deg1_ref, x_ref, ulo_ref, uhi_ref, dinv_ref):
    deg = deg0_ref[:, 0:1] + deg1_ref[:, 0:1] + 1.0
    dinv = lax.rsqrt(deg)
    dinv_ref[...] = dinv
    ulo_ref[...] = x_ref[:, :_F] * dinv
    uhi_ref[...] = x_ref[:, _F:] * dinv


def _prep_call(deg0, deg1, x):
    return pl.pallas_call(
        _prep_body,
        grid=(_GRID,),
        in_specs=[
            pl.BlockSpec((_R, 16), lambda r: (r, 0)),
            pl.BlockSpec((_R, 16), lambda r: (r, 0)),
            pl.BlockSpec((_R, 2 * _F), lambda r: (r, 0)),
        ],
        out_specs=[
            pl.BlockSpec((_R, _F), lambda r: (r, 0)),
            pl.BlockSpec((_R, _F), lambda r: (r, 0)),
            pl.BlockSpec((_R, 1), lambda r: (r, 0)),
        ],
        out_shape=[
            jax.ShapeDtypeStruct((_N, _F), jnp.float32),
            jax.ShapeDtypeStruct((_N, _F), jnp.float32),
            jax.ShapeDtypeStruct((_N, 1), jnp.float32),
        ],
    )(deg0, deg1, x)


def _mid_body(slo_ref, shi_ref, ulo_ref, uhi_ref, dinv_ref, w1_ref, b1_ref,
              w2_ref, olo_ref, ohi_ref):
    dv = dinv_ref[...]
    alo = (slo_ref[...] + ulo_ref[...]) * dv
    ahi = (shi_ref[...] + uhi_ref[...]) * dv
    h = jnp.dot(alo, w1_ref[:_F, :], preferred_element_type=jnp.float32)
    h = h + jnp.dot(ahi, w1_ref[_F:, :], preferred_element_type=jnp.float32)
    h = jnp.maximum(h + b1_ref[...], 0.0)
    g = jnp.dot(h, w2_ref[...], preferred_element_type=jnp.float32)
    olo_ref[...] = g[:, :_F] * dv
    ohi_ref[...] = g[:, _F:] * dv


def _mid_call(slo, shi, ulo, uhi, dinv, W1, b1, W2):
    return pl.pallas_call(
        _mid_body,
        grid=(_GRID,),
        in_specs=[
            pl.BlockSpec((_R, _F), lambda r: (r, 0)),
            pl.BlockSpec((_R, _F), lambda r: (r, 0)),
            pl.BlockSpec((_R, _F), lambda r: (r, 0)),
            pl.BlockSpec((_R, _F), lambda r: (r, 0)),
            pl.BlockSpec((_R, 1), lambda r: (r, 0)),
            pl.BlockSpec((2 * _F, 512), lambda r: (0, 0)),
            pl.BlockSpec((1, 512), lambda r: (0, 0)),
            pl.BlockSpec((512, 2 * _F), lambda r: (0, 0)),
        ],
        out_specs=[
            pl.BlockSpec((_R, _F), lambda r: (r, 0)),
            pl.BlockSpec((_R, _F), lambda r: (r, 0)),
        ],
        out_shape=[
            jax.ShapeDtypeStruct((_N, _F), jnp.float32),
            jax.ShapeDtypeStruct((_N, _F), jnp.float32),
        ],
    )(slo, shi, ulo, uhi, dinv, W1, b1, W2)


def _final_body(slo_ref, shi_ref, ulo_ref, uhi_ref, dinv_ref, b2_ref,
                out_ref):
    dv = dinv_ref[...]
    out_ref[:, :_F] = jax.nn.sigmoid((slo_ref[...] + ulo_ref[...]) * dv
                                     + b2_ref[:, :_F])
    out_ref[:, _F:] = jax.nn.sigmoid((shi_ref[...] + uhi_ref[...]) * dv
                                     + b2_ref[:, _F:])


def _final_call(slo, shi, ulo, uhi, dinv, b2):
    return pl.pallas_call(
        _final_body,
        grid=(_GRID,),
        in_specs=[
            pl.BlockSpec((_R, _F), lambda r: (r, 0)),
            pl.BlockSpec((_R, _F), lambda r: (r, 0)),
            pl.BlockSpec((_R, _F), lambda r: (r, 0)),
            pl.BlockSpec((_R, _F), lambda r: (r, 0)),
            pl.BlockSpec((_R, 1), lambda r: (r, 0)),
            pl.BlockSpec((1, 2 * _F), lambda r: (0, 0)),
        ],
        out_specs=pl.BlockSpec((_R, 2 * _F), lambda r: (r, 0)),
        out_shape=jax.ShapeDtypeStruct((_N, 2 * _F), jnp.float32),
    )(slo, shi, ulo, uhi, dinv, b2)


def kernel(x, edge_index, W1, b1, W2, b2):
    src = edge_index[0].astype(jnp.int32)
    dst = edge_index[1].astype(jnp.int32)

    deg0, deg1 = _deg_kernel(dst)
    ulo, uhi, dinv = _prep_call(deg0, deg1, x)
    s1lo, s1hi = _scatter_kernel(ulo, uhi, src, dst)
    u2lo, u2hi = _mid_call(s1lo, s1hi, ulo, uhi, dinv, W1,
                           b1.reshape(1, -1), W2)
    s2lo, s2hi = _scatter_kernel(u2lo, u2hi, src, dst)
    return _final_call(s2lo, s2hi, u2lo, u2hi, dinv, b2.reshape(1, -1))
